# Initial kernel scaffold; baseline (speedup 1.0000x reference)
import jax
import jax.numpy as jnp
from jax.experimental import pallas as pl


def kernel(feat, loc, embed_table, G_w, agg_w, agg_b, edge_index, inter_ids):
    raise NotImplementedError("write your pallas kernel here")


# speedup vs baseline: 1.0951x; 1.0951x over previous
import functools

import jax
import jax.numpy as jnp
from jax.experimental import pallas as pl
from jax.experimental.pallas import tpu as pltpu

N_NODES = 10000
D = 128
NBUCKETS = 33
ROW_BLK = 400


def _combine_body(s1_ref, s2_ref, c_ref, indeg_ref, w1_ref, w2_ref, b_ref,
                  out_ref):
    d0 = jax.lax.rsqrt(jnp.maximum(indeg_ref[...], 1.0))
    acc = jnp.dot(s1_ref[...], w1_ref[...],
                  preferred_element_type=jnp.float32)
    acc += jnp.dot(s2_ref[...], w2_ref[...],
                   preferred_element_type=jnp.float32)
    acc += c_ref[...] * b_ref[...]
    out_ref[...] = d0 * acc


def _combine(s1, s2, c, indeg, w1, w2, agg_b):
    n = s1.shape[0]
    grid = (n // ROW_BLK,)
    row = lambda i: (i, 0)
    return pl.pallas_call(
        _combine_body,
        grid=grid,
        in_specs=[
            pl.BlockSpec((ROW_BLK, D), row),
            pl.BlockSpec((ROW_BLK, D), row),
            pl.BlockSpec((ROW_BLK, 1), row),
            pl.BlockSpec((ROW_BLK, 1), row),
            pl.BlockSpec((D, D), lambda i: (0, 0)),
            pl.BlockSpec((D, D), lambda i: (0, 0)),
            pl.BlockSpec((1, D), lambda i: (0, 0)),
        ],
        out_specs=pl.BlockSpec((ROW_BLK, D), row),
        out_shape=jax.ShapeDtypeStruct((n, D), jnp.float32),
    )(s1, s2, c, indeg, w1, w2, agg_b)


def _embed_mm_body(e_ref, g_ref, t_ref):
    t_ref[...] = jnp.dot(e_ref[...], g_ref[...],
                         preferred_element_type=jnp.float32)


def _embed_mm(embed_pad, g_w):
    return pl.pallas_call(
        _embed_mm_body,
        out_shape=jax.ShapeDtypeStruct((embed_pad.shape[0], D), jnp.float32),
    )(embed_pad, g_w)


def _bucketize_sq(d2):
    th = (jnp.arange(1, 33, dtype=jnp.float32) * 0.25) ** 2
    return jnp.sum(d2[..., None] > th, axis=-1).astype(jnp.int32)


def kernel(feat, loc, embed_table, G_w, agg_w, agg_b, edge_index, inter_ids):
    src = edge_index[0]
    dst = edge_index[1]
    n, d = feat.shape
    num = inter_ids.shape[1]

    emb_pad = jnp.pad(embed_table, ((0, 7), (0, 0)))
    t_tab = _embed_mm(emb_pad, G_w)

    outdeg = jnp.zeros((n,), jnp.float32).at[src].add(1.0)
    indeg = jnp.zeros((n,), jnp.float32).at[dst].add(1.0)
    d2v = jax.lax.rsqrt(jnp.maximum(outdeg, 1.0))

    loc_src = loc[src]
    diff1 = loc[dst] - loc_src
    dist1_sq = jnp.sum(diff1 * diff1, axis=1)
    b1 = _bucketize_sq(dist1_sq)

    inter_pos = loc[inter_ids]
    diff_ = loc_src[:, None, :] - inter_pos
    dist_sq = jnp.sum(diff_ * diff_, axis=-1)
    b_ = _bucketize_sq(dist_sq)

    coef = d2v[src]
    u = coef[:, None] * t_tab[b1] * feat[src]
    v = (coef / num)[:, None] * jnp.sum(t_tab[b_] * feat[inter_ids], axis=1)

    s1 = jnp.zeros((n, d), jnp.float32).at[dst].add(u)
    s2 = jnp.zeros((n, d), jnp.float32).at[dst].add(v)
    c = jnp.zeros((n,), jnp.float32).at[dst].add(coef)

    return _combine(s1, s2, c[:, None], indeg[:, None],
                    agg_w[:d], agg_w[d:], agg_b[None, :])
